# baseline (speedup 1.0000x reference)
import functools

import jax
import jax.numpy as jnp
import numpy as np
from jax.experimental import pallas as pl
from jax.experimental.pallas import tpu as pltpu

N_BINS = 15
N_ROWS = 65536
N_COLS = 1000
BLOCK_ROWS = 512

_BOUNDS = np.linspace(0.0, 1.0, N_BINS + 1).astype(np.float32)
_LOWERS = np.full((1, 128), 2.0, np.float32)
_LOWERS[0, :N_BINS] = _BOUNDS[:-1]
_UPPERS = np.full((1, 128), 3.0, np.float32)
_UPPERS[0, :N_BINS] = _BOUNDS[1:]


def _ece_kernel(scale_ref, bounds_ref, logits_ref, labels_ref, out_ref, acc_ref):
    i = pl.program_id(0)

    @pl.when(i == 0)
    def _init():
        acc_ref[...] = jnp.zeros_like(acc_ref)

    x = logits_ref[...] * scale_ref[0]
    col = jax.lax.broadcasted_iota(jnp.int32, x.shape, 1)
    valid = col < N_COLS
    neg_inf = jnp.float32(-jnp.inf)
    xm = jnp.where(valid, x, neg_inf)
    m = jnp.max(xm, axis=1, keepdims=True)
    e = jnp.where(valid, jnp.exp(xm - m), 0.0)
    s = jnp.sum(e, axis=1, keepdims=True)
    conf = 1.0 / s
    pred = jnp.min(jnp.where(xm == m, col, N_COLS), axis=1, keepdims=True)
    acc = (pred == labels_ref[...]).astype(jnp.float32)

    lowers = bounds_ref[0:1, :]
    uppers = bounds_ref[1:2, :]
    in_bin = ((conf > lowers) & (conf <= uppers)).astype(jnp.float32)
    count_p = jnp.sum(in_bin, axis=0, keepdims=True)
    acc_p = jnp.sum(acc * in_bin, axis=0, keepdims=True)
    conf_p = jnp.sum(conf * in_bin, axis=0, keepdims=True)
    acc_ref[0:1, :] += count_p
    acc_ref[1:2, :] += acc_p
    acc_ref[2:3, :] += conf_p

    @pl.when(i == pl.num_programs(0) - 1)
    def _finish():
        count = acc_ref[0:1, :]
        acc_sum = acc_ref[1:2, :]
        conf_sum = acc_ref[2:3, :]
        safe = jnp.maximum(count, 1.0)
        contrib = jnp.abs(conf_sum / safe - acc_sum / safe) * (count / N_ROWS)
        contrib = jnp.where(count > 0.0, contrib, 0.0)
        out_ref[...] = jnp.sum(contrib, axis=(0, 1), keepdims=True)


@jax.jit
def _ece(logits, labels, scale, bounds):
    labels2 = labels.astype(jnp.int32).reshape(N_ROWS, 1)
    grid = N_ROWS // BLOCK_ROWS
    out = pl.pallas_call(
        _ece_kernel,
        grid=(grid,),
        in_specs=[
            pl.BlockSpec(memory_space=pltpu.SMEM),
            pl.BlockSpec((2, 128), lambda i: (0, 0)),
            pl.BlockSpec((BLOCK_ROWS, N_COLS), lambda i: (i, 0)),
            pl.BlockSpec((BLOCK_ROWS, 1), lambda i: (i, 0)),
        ],
        out_specs=pl.BlockSpec((1, 1), lambda i: (0, 0)),
        out_shape=jax.ShapeDtypeStruct((1, 1), jnp.float32),
        scratch_shapes=[pltpu.VMEM((8, 128), jnp.float32)],
    )(scale, bounds, logits, labels2)
    return out.reshape(1)


def kernel(logits, labels, t_opt):
    t = jnp.asarray(t_opt, jnp.float32)
    scale = jnp.where(t == 0.0, 1.0, 1.0 / t).reshape(1).astype(jnp.float32)
    bounds = jnp.asarray(np.stack([_LOWERS[0], _UPPERS[0]]))
    return _ece(logits, labels, scale, bounds)
